# baseline (speedup 1.0000x reference)
import functools

import jax
import jax.numpy as jnp
import numpy as np
from jax.experimental import pallas as pl
from jax.experimental.pallas import tpu as pltpu


def _gelu(v):
    return 0.5 * v * (1.0 + jax.lax.erf(v * np.float32(0.7071067811865476)))


def _fused_body(x_ref, w1_ref, b1_ref, gw2_ref, gb2_ref, sel_ref, w2_ref,
                o_ref, *, E, H, NL):
    bf16 = jnp.bfloat16
    xt = x_ref[...]
    y = jnp.dot(xt.astype(bf16), w1_ref[...],
                preferred_element_type=jnp.float32) + b1_ref[...]
    h = _gelu(y[:, :H])
    logits = jnp.dot(h.astype(bf16), gw2_ref[...],
                     preferred_element_type=jnp.float32) + gb2_ref[...]
    probs = jax.nn.softmax(logits, axis=-1)
    nl = jnp.concatenate(
        [_gelu(y[:, H:H + NL - 2]),
         jax.nn.sigmoid(y[:, H + NL - 2:H + NL])], axis=1)
    scale = jnp.dot(probs, sel_ref[...],
                    preferred_element_type=jnp.float32)
    z = jnp.concatenate([nl * scale, probs], axis=1)
    o = jnp.dot(z.astype(bf16), w2_ref[...],
                preferred_element_type=jnp.float32)
    o_ref[...] = xt * (1.0 + o[:, :E]) + o[:, E:]


def kernel(x, reglu_u, reglu_a, reglu_b, reglu_bias, film_dw, film_db,
           film_uw, film_ub, p2_w, p2_v, p2_alpha, p2_b, p2_bias, p4_w, p4_v,
           p4_alpha, p4_b, p4_bias, gate_w1, gate_b1, gate_w2, gate_b2,
           expert_bias, post_mix_alpha):
    B, S, E = x.shape
    H = gate_w1.shape[1]
    R = film_dw.shape[-1]
    r2 = p2_w.shape[1]
    r4 = p4_w.shape[1]
    M = gate_w2.shape[1]
    NL = 2 * R + 2 * r2 + 2 * r4 + 2
    K1 = H + NL
    K1P = 128 * ((K1 + 127) // 128)

    f32 = jnp.float32

    w1 = jnp.concatenate([
        gate_w1,
        film_dw[0], film_dw[1],
        p2_w[0].T, p2_w[1].T,
        p4_w[0].T, p4_w[1].T,
        reglu_u[0][:, None], reglu_u[1][:, None],
    ], axis=1)
    w1 = jnp.pad(w1, ((0, 0), (0, K1P - K1)))
    b1 = jnp.concatenate([
        gate_b1,
        film_db[0], film_db[1],
        p2_b[0], p2_b[1],
        p4_b[0], p4_b[1],
        reglu_b[0:1], reglu_b[1:2],
    ])
    b1 = jnp.pad(b1, (0, K1P - K1))[None, :]

    gb2 = (gate_b2 + expert_bias)[None, :]

    sel_np = np.zeros((M, NL), dtype=np.float32)
    c = 0
    sel_np[1, c:c + R] = 1.0; c += R
    sel_np[5, c:c + R] = 1.0; c += R
    sel_np[2, c:c + r2] = 1.0; c += r2
    sel_np[6, c:c + r2] = 1.0; c += r2
    sel_np[3, c:c + r4] = 1.0; c += r4
    sel_np[7, c:c + r4] = 1.0; c += r4
    sel_np[0, c] = 1.0; c += 1
    sel_np[4, c] = 1.0; c += 1
    sel = jnp.asarray(sel_np)

    zE = jnp.zeros((1, E), dtype=f32)
    w2 = jnp.concatenate([
        film_uw[0],
        film_uw[1],
        jnp.concatenate([jnp.zeros((r2, E), f32),
                         p2_alpha[0][:, None] * p2_v[0]], axis=1),
        jnp.concatenate([jnp.zeros((r2, E), f32),
                         p2_alpha[1][:, None] * p2_v[1]], axis=1),
        jnp.concatenate([jnp.zeros((r4, E), f32),
                         p4_alpha[0][:, None] * p4_v[0]], axis=1),
        jnp.concatenate([jnp.zeros((r4, E), f32),
                         p4_alpha[1][:, None] * p4_v[1]], axis=1),
        jnp.concatenate([reglu_a[0][None, :], zE], axis=1),
        jnp.concatenate([reglu_a[1][None, :], zE], axis=1),
        jnp.concatenate([zE, reglu_bias[0][None, :]], axis=1),
        film_ub[0][None, :],
        jnp.concatenate([zE, p2_bias[0][None, :]], axis=1),
        jnp.concatenate([zE, p4_bias[0][None, :]], axis=1),
        jnp.concatenate([zE, reglu_bias[1][None, :]], axis=1),
        film_ub[1][None, :],
        jnp.concatenate([zE, p2_bias[1][None, :]], axis=1),
        jnp.concatenate([zE, p4_bias[1][None, :]], axis=1),
    ], axis=0) * post_mix_alpha

    N = B * S
    T = 2048
    x2 = x.reshape(N, E)
    w1 = w1.astype(jnp.bfloat16)
    gw2 = gate_w2.astype(jnp.bfloat16)
    w2 = w2.astype(jnp.bfloat16)

    body = functools.partial(_fused_body, E=E, H=H, NL=NL)
    out = pl.pallas_call(
        body,
        grid=(N // T,),
        in_specs=[
            pl.BlockSpec((T, E), lambda i: (i, 0)),
            pl.BlockSpec((E, K1P), lambda i: (0, 0)),
            pl.BlockSpec((1, K1P), lambda i: (0, 0)),
            pl.BlockSpec((H, M), lambda i: (0, 0)),
            pl.BlockSpec((1, M), lambda i: (0, 0)),
            pl.BlockSpec((M, NL), lambda i: (0, 0)),
            pl.BlockSpec((NL + M, 2 * E), lambda i: (0, 0)),
        ],
        out_specs=pl.BlockSpec((T, E), lambda i: (i, 0)),
        out_shape=jax.ShapeDtypeStruct((N, E), f32),
        compiler_params=pltpu.CompilerParams(
            dimension_semantics=("parallel",)),
    )(x2, w1, b1, gw2, gb2, sel, w2)
    return out.reshape(B, S, E)


# speedup vs baseline: 1.1104x; 1.1104x over previous
import functools

import jax
import jax.numpy as jnp
import numpy as np
from jax.experimental import pallas as pl
from jax.experimental.pallas import tpu as pltpu


def _gelu(v):
    return 0.5 * v * (1.0 + jax.lax.erf(v * np.float32(0.7071067811865476)))


def _fused_body(x_ref, gw1_ref, gw2_ref, gb1_ref, gb2_ref, tr_ref, fdw_ref,
                fdb_ref, fuw_ref, fub_ref, ra_ref, rbias_ref, pv_ref,
                p2b_ref, p4b_ref, sm_ref, sel_ref, alpha_ref,
                o_ref, w1s, b1s, w2s, *, E, H, R, NL, K1P, K2):
    bf16 = jnp.bfloat16
    f32 = jnp.float32
    i = pl.program_id(0)

    @pl.when(i == 0)
    def _pack():
        alpha = alpha_ref[0, 0]
        w1s[...] = jnp.concatenate([
            gw1_ref[...],
            fdw_ref[0:E, :], fdw_ref[E:2 * E, :],
            tr_ref[...],
            jnp.zeros((E, K1P - H - NL), f32),
        ], axis=1).astype(bf16)
        b1s[...] = jnp.concatenate([
            gb1_ref[...],
            fdb_ref[0:1, :], fdb_ref[1:2, :],
            sm_ref[...],
            jnp.zeros((1, K1P - H - NL), f32),
        ], axis=1)
        zE1 = jnp.zeros((1, E), f32)
        w2s[...] = (jnp.concatenate([
            fuw_ref[...],
            jnp.concatenate([jnp.zeros((12, E), f32), pv_ref[...]], axis=1),
            jnp.concatenate([ra_ref[...], jnp.zeros((2, E), f32)], axis=1),
            jnp.concatenate([zE1, rbias_ref[0:1, :]], axis=1),
            fub_ref[0:1, :],
            jnp.concatenate([zE1, p2b_ref[0:1, :]], axis=1),
            jnp.concatenate([zE1, p4b_ref[0:1, :]], axis=1),
            jnp.concatenate([zE1, rbias_ref[1:2, :]], axis=1),
            fub_ref[1:2, :],
            jnp.concatenate([zE1, p2b_ref[1:2, :]], axis=1),
            jnp.concatenate([zE1, p4b_ref[1:2, :]], axis=1),
        ], axis=0) * alpha).astype(bf16)

    xt = x_ref[...]
    y = jnp.dot(xt.astype(bf16), w1s[...],
                preferred_element_type=f32) + b1s[...]
    h = _gelu(y[:, :H])
    logits = jnp.dot(h.astype(bf16), gw2_ref[...].astype(bf16),
                     preferred_element_type=f32) + gb2_ref[...]
    probs = jax.nn.softmax(logits, axis=-1)
    nl = jnp.concatenate(
        [_gelu(y[:, H:H + NL - 2]),
         jax.nn.sigmoid(y[:, H + NL - 2:H + NL])], axis=1)
    scale = jnp.dot(probs, sel_ref[...],
                    preferred_element_type=f32)
    z = jnp.concatenate([nl * scale, probs], axis=1)
    o = jnp.dot(z.astype(bf16), w2s[...],
                preferred_element_type=f32)
    o_ref[...] = xt * (1.0 + o[:, :E]) + o[:, E:]


def kernel(x, reglu_u, reglu_a, reglu_b, reglu_bias, film_dw, film_db,
           film_uw, film_ub, p2_w, p2_v, p2_alpha, p2_b, p2_bias, p4_w, p4_v,
           p4_alpha, p4_b, p4_bias, gate_w1, gate_b1, gate_w2, gate_b2,
           expert_bias, post_mix_alpha):
    B, S, E = x.shape
    H = gate_w1.shape[1]
    R = film_dw.shape[-1]
    r2 = p2_w.shape[1]
    r4 = p4_w.shape[1]
    M = gate_w2.shape[1]
    NL = 2 * R + 2 * r2 + 2 * r4 + 2
    K1 = H + NL
    K1P = 128 * ((K1 + 127) // 128)
    K2 = NL + M

    f32 = jnp.float32
    bf16 = jnp.bfloat16

    tr = jnp.concatenate([
        p2_w.reshape(2 * r2, E), p4_w.reshape(2 * r4, E), reglu_u,
    ], axis=0).T
    pv = jnp.concatenate([
        p2_alpha.reshape(2 * r2, 1) * p2_v.reshape(2 * r2, E),
        p4_alpha.reshape(2 * r4, 1) * p4_v.reshape(2 * r4, E),
    ], axis=0)
    smalls = jnp.concatenate([
        p2_b.reshape(-1), p4_b.reshape(-1), reglu_b,
    ])[None, :]
    gb2 = (gate_b2 + expert_bias)[None, :]
    alpha2d = post_mix_alpha.reshape(1, 1)

    sel_np = np.zeros((M, NL), dtype=np.float32)
    c = 0
    sel_np[1, c:c + R] = 1.0; c += R
    sel_np[5, c:c + R] = 1.0; c += R
    sel_np[2, c:c + r2] = 1.0; c += r2
    sel_np[6, c:c + r2] = 1.0; c += r2
    sel_np[3, c:c + r4] = 1.0; c += r4
    sel_np[7, c:c + r4] = 1.0; c += r4
    sel_np[0, c] = 1.0; c += 1
    sel_np[4, c] = 1.0; c += 1
    sel = jnp.asarray(sel_np)

    N = B * S
    T = 2048
    x2 = x.reshape(N, E)

    def full(shape):
        nzero = len(shape)
        return pl.BlockSpec(shape, lambda i, _n=nzero: (0,) * _n)

    body = functools.partial(_fused_body, E=E, H=H, R=R, NL=NL, K1P=K1P,
                             K2=K2)
    out = pl.pallas_call(
        body,
        grid=(N // T,),
        in_specs=[
            pl.BlockSpec((T, E), lambda i: (i, 0)),
            full((E, H)),
            full((H, M)),
            full((1, H)),
            full((1, M)),
            full((E, 2 * r2 + 2 * r4 + 2)),
            full((2 * E, R)),
            full((2, R)),
            full((2 * R, 2 * E)),
            full((2, 2 * E)),
            full((2, E)),
            full((2, E)),
            full((12, E)),
            full((2, E)),
            full((2, E)),
            full((1, 2 * r2 + 2 * r4 + 2)),
            full((M, NL)),
            pl.BlockSpec(memory_space=pltpu.SMEM),
        ],
        out_specs=pl.BlockSpec((T, E), lambda i: (i, 0)),
        out_shape=jax.ShapeDtypeStruct((N, E), f32),
        scratch_shapes=[
            pltpu.VMEM((E, K1P), bf16),
            pltpu.VMEM((1, K1P), f32),
            pltpu.VMEM((K2, 2 * E), bf16),
        ],
    )(x2, gate_w1, gate_w2, gate_b1[None, :], gb2, tr,
      film_dw.reshape(2 * E, R), film_db, film_uw.reshape(2 * R, 2 * E),
      film_ub, reglu_a, reglu_bias, pv, p2_bias, p4_bias, smalls, sel,
      alpha2d)
    return out.reshape(B, S, E)


# speedup vs baseline: 1.1999x; 1.0806x over previous
import functools

import jax
import jax.numpy as jnp
import numpy as np
from jax.experimental import pallas as pl
from jax.experimental.pallas import tpu as pltpu


def _gelu(v):
    return 0.5 * v * (1.0 + jax.lax.erf(v * np.float32(0.7071067811865476)))


def _fused_body(x_ref, gw1_ref, gw2_ref, gb1_ref, gb2_ref, eb_ref,
                p2w_ref, p4w_ref, ru_ref, rb_ref, fdw_ref, fdb_ref, fuw_ref,
                fub_ref,
                ra_ref, rbias_ref, p2v_ref, p4v_ref, p2b_ref, p4b_ref,
                pbias2_ref, pbias4_ref, sel_ref,
                p2a_ref, p4a_ref, alpha_ref,
                o_ref, w1s, gw2s, b2s, w2s, *, E, H, R, NL, K1P, K2):
    bf16 = jnp.bfloat16
    f32 = jnp.float32
    i = pl.program_id(0)

    @pl.when(i == 0)
    def _pack():
        alpha = alpha_ref[0, 0]
        cat16 = jnp.concatenate([
            p2w_ref[...], p4w_ref[...], ru_ref[...],
            jnp.zeros((2, E), f32),
        ], axis=0)
        w1s[...] = jnp.concatenate([
            gw1_ref[...],
            fdw_ref[0:E, :], fdw_ref[E:2 * E, :],
            cat16.T,
            jnp.zeros((E, K1P - H - 2 * R - 16), f32),
        ], axis=1).astype(bf16)
        gw2s[...] = gw2_ref[...].astype(bf16)
        b2s[...] = jnp.concatenate([
            fdb_ref[0:1, :], fdb_ref[1:2, :],
            p2b_ref[0:1, :], p2b_ref[1:2, :],
            p4b_ref[0:1, :], p4b_ref[1:2, :],
            rb_ref[...],
        ], axis=1)
        zE1 = jnp.zeros((1, E), f32)
        pv_rows = (
            [p2v_ref[j:j + 1, :] * p2a_ref[j // 2, j % 2] for j in range(4)]
            + [p4v_ref[j:j + 1, :] * p4a_ref[j // 4, j % 4] for j in range(8)]
        )
        w2s[...] = (jnp.concatenate([
            fuw_ref[...],
            jnp.concatenate(
                [jnp.zeros((12, E), f32),
                 jnp.concatenate(pv_rows, axis=0)], axis=1),
            jnp.concatenate([ra_ref[...], jnp.zeros((2, E), f32)], axis=1),
            jnp.concatenate([zE1, rbias_ref[0:1, :]], axis=1),
            fub_ref[0:1, :],
            jnp.concatenate([zE1, pbias2_ref[0:1, :]], axis=1),
            jnp.concatenate([zE1, pbias4_ref[0:1, :]], axis=1),
            jnp.concatenate([zE1, rbias_ref[1:2, :]], axis=1),
            fub_ref[1:2, :],
            jnp.concatenate([zE1, pbias2_ref[1:2, :]], axis=1),
            jnp.concatenate([zE1, pbias4_ref[1:2, :]], axis=1),
        ], axis=0) * alpha).astype(bf16)

    xt = x_ref[...]
    y = jnp.dot(xt.astype(bf16), w1s[...], preferred_element_type=f32)
    h = _gelu(y[:, :H] + gb1_ref[...])
    logits = (jnp.dot(h.astype(bf16), gw2s[...], preferred_element_type=f32)
              + (gb2_ref[...] + eb_ref[...]))
    probs = jax.nn.softmax(logits, axis=-1)
    nlp = y[:, H:H + NL] + b2s[...]
    nl = jnp.concatenate(
        [_gelu(nlp[:, :NL - 2]), jax.nn.sigmoid(nlp[:, NL - 2:])],
        axis=1)
    scale = jnp.dot(probs, sel_ref[...], preferred_element_type=f32)
    z = jnp.concatenate([nl * scale, probs], axis=1)
    o = jnp.dot(z.astype(bf16), w2s[...], preferred_element_type=f32)
    o_ref[...] = xt * (1.0 + o[:, :E]) + o[:, E:]


def kernel(x, reglu_u, reglu_a, reglu_b, reglu_bias, film_dw, film_db,
           film_uw, film_ub, p2_w, p2_v, p2_alpha, p2_b, p2_bias, p4_w, p4_v,
           p4_alpha, p4_b, p4_bias, gate_w1, gate_b1, gate_w2, gate_b2,
           expert_bias, post_mix_alpha):
    B, S, E = x.shape
    H = gate_w1.shape[1]
    R = film_dw.shape[-1]
    r2 = p2_w.shape[1]
    r4 = p4_w.shape[1]
    M = gate_w2.shape[1]
    NL = 2 * R + 2 * r2 + 2 * r4 + 2
    K1P = 384
    K2 = NL + M

    f32 = jnp.float32
    bf16 = jnp.bfloat16

    sel_np = np.zeros((M, NL), dtype=np.float32)
    c = 0
    sel_np[1, c:c + R] = 1.0; c += R
    sel_np[5, c:c + R] = 1.0; c += R
    sel_np[2, c:c + r2] = 1.0; c += r2
    sel_np[6, c:c + r2] = 1.0; c += r2
    sel_np[3, c:c + r4] = 1.0; c += r4
    sel_np[7, c:c + r4] = 1.0; c += r4
    sel_np[0, c] = 1.0; c += 1
    sel_np[4, c] = 1.0; c += 1
    sel = jnp.asarray(sel_np)

    N = B * S
    T = 2048
    x2 = x.reshape(N, E)

    def full(shape):
        n = len(shape)
        return pl.BlockSpec(shape, lambda i, _n=n: (0,) * _n)

    smem = pl.BlockSpec(memory_space=pltpu.SMEM)
    body = functools.partial(_fused_body, E=E, H=H, R=R, NL=NL, K1P=K1P,
                             K2=K2)
    out = pl.pallas_call(
        body,
        grid=(N // T,),
        in_specs=[
            pl.BlockSpec((T, E), lambda i: (i, 0)),
            full((E, H)),
            full((H, M)),
            full((1, H)),
            full((1, M)),
            full((1, M)),
            full((2 * r2, E)),
            full((2 * r4, E)),
            full((2, E)),
            full((1, 2)),
            full((2 * E, R)),
            full((2, R)),
            full((2 * R, 2 * E)),
            full((2, 2 * E)),
            full((2, E)),
            full((2, E)),
            full((2 * r2, E)),
            full((2 * r4, E)),
            full((2, r2)),
            full((2, r4)),
            full((2, E)),
            full((2, E)),
            full((M, NL)),
            smem,
            smem,
            smem,
        ],
        out_specs=pl.BlockSpec((T, E), lambda i: (i, 0)),
        out_shape=jax.ShapeDtypeStruct((N, E), f32),
        scratch_shapes=[
            pltpu.VMEM((E, K1P), bf16),
            pltpu.VMEM((H, M), bf16),
            pltpu.VMEM((1, NL), f32),
            pltpu.VMEM((K2, 2 * E), bf16),
        ],
    )(x2, gate_w1, gate_w2, gate_b1[None, :], gate_b2[None, :],
      expert_bias[None, :], p2_w.reshape(2 * r2, E), p4_w.reshape(2 * r4, E),
      reglu_u, reglu_b[None, :], film_dw.reshape(2 * E, R), film_db,
      film_uw.reshape(2 * R, 2 * E), film_ub, reglu_a, reglu_bias,
      p2_v.reshape(2 * r2, E), p4_v.reshape(2 * r4, E), p2_b, p4_b,
      p2_bias, p4_bias, sel, p2_alpha, p4_alpha, post_mix_alpha.reshape(1, 1))
    return out.reshape(B, S, E)
